# baseline (speedup 1.0000x reference)
import functools

import jax
import jax.numpy as jnp
from jax import lax
from jax.experimental import pallas as pl
from jax.experimental.pallas import tpu as pltpu
from jax.experimental.pallas import tpu_sc as plsc

B = 16384
E = 64
H1 = 256
H2 = 128
NC = 2
NS = 16
NW = NC * NS
BPW = B // NW
CHUNK = 128
NCHUNK = BPW // CHUNK
PW = 2 * E
PBLK = 1024

_sc_mesh = plsc.VectorSubcoreMesh(core_axis_name="c", subcore_axis_name="s")


def _pack_body(x_ref, o_ref):
    t = jnp.swapaxes(x_ref[...], 0, 1)
    d = jnp.concatenate([t, t], axis=1).astype(jnp.bfloat16)
    o_ref[...] = pltpu.bitcast(d, jnp.int32)


def _make_packer(n_rows):
    return pl.pallas_call(
        _pack_body,
        grid=(pl.cdiv(n_rows, PBLK),),
        in_specs=[pl.BlockSpec((E, PBLK), lambda i: (0, i))],
        out_specs=pl.BlockSpec((PBLK // 2, PW), lambda i: (i, 0)),
        out_shape=jax.ShapeDtypeStruct((n_rows // 2, PW), jnp.int32),
    )


_pack_user = _make_packer(1000000)
_pack_movie = _make_packer(100000)


@functools.partial(
    pl.kernel,
    out_type=(
        jax.ShapeDtypeStruct((B, PW), jnp.int32),
        jax.ShapeDtypeStruct((B, PW), jnp.int32),
    ),
    mesh=_sc_mesh,
    scratch_types=[
        pltpu.VMEM((NCHUNK, CHUNK), jnp.int32),
        pltpu.VMEM((NCHUNK, CHUNK), jnp.int32),
        pltpu.VMEM((BPW, PW), jnp.int32),
        pltpu.SemaphoreType.DMA,
    ],
)
def _gather_sc(puid_hbm, pmid_hbm, utab_hbm, mtab_hbm, u_out, m_out,
               uidx_v, midx_v, rows_v, sem):
    wid = lax.axis_index("s") * NC + lax.axis_index("c")
    base = wid * BPW
    for j in range(NCHUNK):
        pltpu.sync_copy(puid_hbm.at[pl.ds(base + j * CHUNK, CHUNK)], uidx_v.at[j])
        pltpu.sync_copy(pmid_hbm.at[pl.ds(base + j * CHUNK, CHUNK)], midx_v.at[j])
    ucopies = [
        pltpu.async_copy(utab_hbm.at[uidx_v.at[j]],
                         rows_v.at[pl.ds(j * CHUNK, CHUNK)], sem)
        for j in range(NCHUNK)
    ]
    for c in ucopies:
        c.wait()
    pltpu.sync_copy(rows_v, u_out.at[pl.ds(base, BPW)])
    mcopies = [
        pltpu.async_copy(mtab_hbm.at[midx_v.at[j]],
                         rows_v.at[pl.ds(j * CHUNK, CHUNK)], sem)
        for j in range(NCHUNK)
    ]
    for c in mcopies:
        c.wait()
    pltpu.sync_copy(rows_v, m_out.at[pl.ds(base, BPW)])


BB = 1024
NBLK = B // BB
_HI = -65536


def _unpack_half(words, parity_is_odd):
    bits = jnp.where(parity_is_odd, words & _HI, words << 16)
    return lax.bitcast_convert_type(bits, jnp.float32)


def _mlp_body(u2_ref, m2_ref, uid_ref, mid_ref, w1a_ref, w1b_ref, b1_ref,
              w2_ref, b2_ref, w3_ref, b3_ref, out_ref):
    upar = (uid_ref[...] % 2)[:, None] == 1
    mpar = (mid_ref[...] % 2)[:, None] == 1
    u = _unpack_half(u2_ref[...][:, :E], upar)
    m = _unpack_half(m2_ref[...][:, :E], mpar)
    h = jnp.dot(u, w1a_ref[...], preferred_element_type=jnp.float32)
    h = h + jnp.dot(m, w1b_ref[...], preferred_element_type=jnp.float32)
    h = jnp.maximum(h + b1_ref[...], 0.0)
    h = jnp.dot(h, w2_ref[...], preferred_element_type=jnp.float32)
    h = jnp.maximum(h + b2_ref[...], 0.0)
    out_ref[...] = jnp.sum(h * w3_ref[...], axis=1) + b3_ref[0]


_mlp = pl.pallas_call(
    _mlp_body,
    grid=(NBLK,),
    in_specs=[
        pl.BlockSpec((BB, PW), lambda i: (i, 0)),
        pl.BlockSpec((BB, PW), lambda i: (i, 0)),
        pl.BlockSpec((BB,), lambda i: (i,)),
        pl.BlockSpec((BB,), lambda i: (i,)),
        pl.BlockSpec((E, H1), lambda i: (0, 0)),
        pl.BlockSpec((E, H1), lambda i: (0, 0)),
        pl.BlockSpec((1, H1), lambda i: (0, 0)),
        pl.BlockSpec((H1, H2), lambda i: (0, 0)),
        pl.BlockSpec((1, H2), lambda i: (0, 0)),
        pl.BlockSpec((1, H2), lambda i: (0, 0)),
        pl.BlockSpec(memory_space=pltpu.SMEM),
    ],
    out_specs=pl.BlockSpec((BB,), lambda i: (i,)),
    out_shape=jax.ShapeDtypeStruct((B,), jnp.float32),
)


def kernel(user_ids, movie_ids, user_table, movie_table, W1, b1, W2, b2, W3, b3):
    uids = user_ids.astype(jnp.int32)
    mids = movie_ids.astype(jnp.int32)
    upak = _pack_user(user_table.T)
    mpak = _pack_movie(movie_table.T)
    u2, m2 = _gather_sc(uids >> 1, mids >> 1, upak, mpak)
    return _mlp(u2, m2, uids, mids, W1[:E], W1[E:], b1.reshape(1, H1), W2,
                b2.reshape(1, H2), W3.reshape(1, H2), b3)


# speedup vs baseline: 2.4648x; 2.4648x over previous
import functools

import jax
import jax.numpy as jnp
from jax import lax
from jax.experimental import pallas as pl
from jax.experimental.pallas import tpu as pltpu
from jax.experimental.pallas import tpu_sc as plsc

B = 16384
E = 64
H1 = 256
H2 = 128
NC = 2
NS = 16
NW = NC * NS
BPW = B // NW
CHUNK = 128
NCHUNK = BPW // CHUNK
PW = 2 * E
PBLK = 8192

_sc_mesh = plsc.VectorSubcoreMesh(core_axis_name="c", subcore_axis_name="s")


def _pack_body(x_ref, o_ref):
    t = jnp.swapaxes(x_ref[...], 0, 1)
    d = jnp.concatenate([t, t], axis=1).astype(jnp.bfloat16)
    o_ref[...] = pltpu.bitcast(d, jnp.int32)


def _make_packer(n_rows):
    return pl.pallas_call(
        _pack_body,
        grid=(pl.cdiv(n_rows, PBLK),),
        in_specs=[pl.BlockSpec((E, PBLK), lambda i: (0, i))],
        out_specs=pl.BlockSpec((PBLK // 2, PW), lambda i: (i, 0)),
        out_shape=jax.ShapeDtypeStruct((n_rows // 2, PW), jnp.int32),
    )


_pack_user = _make_packer(1000000)
_pack_movie = _make_packer(100000)


@functools.partial(
    pl.kernel,
    out_type=(
        jax.ShapeDtypeStruct((B, PW), jnp.int32),
        jax.ShapeDtypeStruct((B, PW), jnp.int32),
    ),
    mesh=_sc_mesh,
    scratch_types=[
        pltpu.VMEM((NCHUNK, CHUNK), jnp.int32),
        pltpu.VMEM((NCHUNK, CHUNK), jnp.int32),
        pltpu.VMEM((BPW, PW), jnp.int32),
        pltpu.SemaphoreType.DMA,
    ],
)
def _gather_sc(puid_hbm, pmid_hbm, utab_hbm, mtab_hbm, u_out, m_out,
               uidx_v, midx_v, rows_v, sem):
    wid = lax.axis_index("s") * NC + lax.axis_index("c")
    base = wid * BPW
    for j in range(NCHUNK):
        pltpu.sync_copy(puid_hbm.at[pl.ds(base + j * CHUNK, CHUNK)], uidx_v.at[j])
        pltpu.sync_copy(pmid_hbm.at[pl.ds(base + j * CHUNK, CHUNK)], midx_v.at[j])
    ucopies = [
        pltpu.async_copy(utab_hbm.at[uidx_v.at[j]],
                         rows_v.at[pl.ds(j * CHUNK, CHUNK)], sem)
        for j in range(NCHUNK)
    ]
    for c in ucopies:
        c.wait()
    pltpu.sync_copy(rows_v, u_out.at[pl.ds(base, BPW)])
    mcopies = [
        pltpu.async_copy(mtab_hbm.at[midx_v.at[j]],
                         rows_v.at[pl.ds(j * CHUNK, CHUNK)], sem)
        for j in range(NCHUNK)
    ]
    for c in mcopies:
        c.wait()
    pltpu.sync_copy(rows_v, m_out.at[pl.ds(base, BPW)])


BB = 1024
NBLK = B // BB
_HI = -65536


def _unpack_half(words, parity_is_odd):
    bits = jnp.where(parity_is_odd, words & _HI, words << 16)
    return lax.bitcast_convert_type(bits, jnp.float32)


def _mlp_body(u2_ref, m2_ref, uid_ref, mid_ref, w1a_ref, w1b_ref, b1_ref,
              w2_ref, b2_ref, w3_ref, b3_ref, out_ref):
    upar = (uid_ref[...] % 2)[:, None] == 1
    mpar = (mid_ref[...] % 2)[:, None] == 1
    u = _unpack_half(u2_ref[...][:, :E], upar)
    m = _unpack_half(m2_ref[...][:, :E], mpar)
    h = jnp.dot(u, w1a_ref[...], preferred_element_type=jnp.float32)
    h = h + jnp.dot(m, w1b_ref[...], preferred_element_type=jnp.float32)
    h = jnp.maximum(h + b1_ref[...], 0.0)
    h = jnp.dot(h, w2_ref[...], preferred_element_type=jnp.float32)
    h = jnp.maximum(h + b2_ref[...], 0.0)
    out_ref[...] = jnp.sum(h * w3_ref[...], axis=1) + b3_ref[0]


_mlp = pl.pallas_call(
    _mlp_body,
    grid=(NBLK,),
    in_specs=[
        pl.BlockSpec((BB, PW), lambda i: (i, 0)),
        pl.BlockSpec((BB, PW), lambda i: (i, 0)),
        pl.BlockSpec((BB,), lambda i: (i,)),
        pl.BlockSpec((BB,), lambda i: (i,)),
        pl.BlockSpec((E, H1), lambda i: (0, 0)),
        pl.BlockSpec((E, H1), lambda i: (0, 0)),
        pl.BlockSpec((1, H1), lambda i: (0, 0)),
        pl.BlockSpec((H1, H2), lambda i: (0, 0)),
        pl.BlockSpec((1, H2), lambda i: (0, 0)),
        pl.BlockSpec((1, H2), lambda i: (0, 0)),
        pl.BlockSpec(memory_space=pltpu.SMEM),
    ],
    out_specs=pl.BlockSpec((BB,), lambda i: (i,)),
    out_shape=jax.ShapeDtypeStruct((B,), jnp.float32),
)


def kernel(user_ids, movie_ids, user_table, movie_table, W1, b1, W2, b2, W3, b3):
    uids = user_ids.astype(jnp.int32)
    mids = movie_ids.astype(jnp.int32)
    upak = _pack_user(user_table.T)
    mpak = _pack_movie(movie_table.T)
    u2, m2 = _gather_sc(uids >> 1, mids >> 1, upak, mpak)
    return _mlp(u2, m2, uids, mids, W1[:E], W1[E:], b1.reshape(1, H1), W2,
                b2.reshape(1, H2), W3.reshape(1, H2), b3)


# speedup vs baseline: 2.4813x; 1.0067x over previous
import functools

import jax
import jax.numpy as jnp
from jax import lax
from jax.experimental import pallas as pl
from jax.experimental.pallas import tpu as pltpu
from jax.experimental.pallas import tpu_sc as plsc

B = 16384
E = 64
H1 = 256
H2 = 128
NC = 2
NS = 16
NW = NC * NS
BPW = B // NW
CHUNK = 128
NCHUNK = BPW // CHUNK
PW = 2 * E
PBLK = 8192

_sc_mesh = plsc.VectorSubcoreMesh(core_axis_name="c", subcore_axis_name="s")


def _pack_body(x_ref, o_ref):
    t = jnp.swapaxes(x_ref[...], 0, 1)
    d = jnp.concatenate([t, t], axis=1).astype(jnp.bfloat16)
    o_ref[...] = pltpu.bitcast(d, jnp.int32)


def _make_packer(n_rows):
    return pl.pallas_call(
        _pack_body,
        grid=(pl.cdiv(n_rows, PBLK),),
        in_specs=[pl.BlockSpec((E, PBLK), lambda i: (0, i))],
        out_specs=pl.BlockSpec((PBLK // 2, PW), lambda i: (i, 0)),
        out_shape=jax.ShapeDtypeStruct((n_rows // 2, PW), jnp.int32),
    )


_pack_user = _make_packer(1000000)
_pack_movie = _make_packer(100000)


def _gather_body(pid_hbm, tab_hbm, out_hbm, idx_v, rows_v, sem):
    wid = lax.axis_index("s") * NC + lax.axis_index("c")
    base = wid * BPW
    for j in range(NCHUNK):
        pltpu.sync_copy(pid_hbm.at[pl.ds(base + j * CHUNK, CHUNK)], idx_v.at[j])
    copies = [
        pltpu.async_copy(tab_hbm.at[idx_v.at[j]],
                         rows_v.at[pl.ds(j * CHUNK, CHUNK)], sem)
        for j in range(NCHUNK)
    ]
    for c in copies:
        c.wait()
    pltpu.sync_copy(rows_v, out_hbm.at[pl.ds(base, BPW)])


def _make_gather():
    return functools.partial(
        pl.kernel,
        out_type=jax.ShapeDtypeStruct((B, PW), jnp.int32),
        mesh=_sc_mesh,
        scratch_types=[
            pltpu.VMEM((NCHUNK, CHUNK), jnp.int32),
            pltpu.VMEM((BPW, PW), jnp.int32),
            pltpu.SemaphoreType.DMA,
        ],
    )(_gather_body)


_gather_user = _make_gather()
_gather_movie = _make_gather()


BB = 2048
NBLK = B // BB
_HI = -65536


def _unpack_half(words, parity_is_odd):
    bits = jnp.where(parity_is_odd, words & _HI, words << 16)
    return lax.bitcast_convert_type(bits, jnp.float32)


def _mlp_body(u2_ref, m2_ref, uid_ref, mid_ref, w1a_ref, w1b_ref, b1_ref,
              w2_ref, b2_ref, w3_ref, b3_ref, out_ref):
    upar = (uid_ref[...] % 2)[:, None] == 1
    mpar = (mid_ref[...] % 2)[:, None] == 1
    u = _unpack_half(u2_ref[...][:, :E], upar)
    m = _unpack_half(m2_ref[...][:, :E], mpar)
    h = jnp.dot(u, w1a_ref[...], preferred_element_type=jnp.float32)
    h = h + jnp.dot(m, w1b_ref[...], preferred_element_type=jnp.float32)
    h = jnp.maximum(h + b1_ref[...], 0.0)
    h = jnp.dot(h, w2_ref[...], preferred_element_type=jnp.float32)
    h = jnp.maximum(h + b2_ref[...], 0.0)
    out_ref[...] = jnp.sum(h * w3_ref[...], axis=1) + b3_ref[0]


_mlp = pl.pallas_call(
    _mlp_body,
    grid=(NBLK,),
    in_specs=[
        pl.BlockSpec((BB, PW), lambda i: (i, 0)),
        pl.BlockSpec((BB, PW), lambda i: (i, 0)),
        pl.BlockSpec((BB,), lambda i: (i,)),
        pl.BlockSpec((BB,), lambda i: (i,)),
        pl.BlockSpec((E, H1), lambda i: (0, 0)),
        pl.BlockSpec((E, H1), lambda i: (0, 0)),
        pl.BlockSpec((1, H1), lambda i: (0, 0)),
        pl.BlockSpec((H1, H2), lambda i: (0, 0)),
        pl.BlockSpec((1, H2), lambda i: (0, 0)),
        pl.BlockSpec((1, H2), lambda i: (0, 0)),
        pl.BlockSpec(memory_space=pltpu.SMEM),
    ],
    out_specs=pl.BlockSpec((BB,), lambda i: (i,)),
    out_shape=jax.ShapeDtypeStruct((B,), jnp.float32),
)


def kernel(user_ids, movie_ids, user_table, movie_table, W1, b1, W2, b2, W3, b3):
    uids = user_ids.astype(jnp.int32)
    mids = movie_ids.astype(jnp.int32)
    mpak = _pack_movie(movie_table.T)
    m2 = _gather_movie(mids >> 1, mpak)
    upak = _pack_user(user_table.T)
    u2 = _gather_user(uids >> 1, upak)
    return _mlp(u2, m2, uids, mids, W1[:E], W1[E:], b1.reshape(1, H1), W2,
                b2.reshape(1, H2), W3.reshape(1, H2), b3)


# speedup vs baseline: 2.4937x; 1.0050x over previous
import functools

import jax
import jax.numpy as jnp
from jax import lax
from jax.experimental import pallas as pl
from jax.experimental.pallas import tpu as pltpu
from jax.experimental.pallas import tpu_sc as plsc

B = 16384
E = 64
H1 = 256
H2 = 128
NC = 2
NS = 16
NW = NC * NS
BPW = B // NW
CHUNK = 128
NCHUNK = BPW // CHUNK
PW = 2 * E
PBLK = 8192

_sc_mesh = plsc.VectorSubcoreMesh(core_axis_name="c", subcore_axis_name="s")


def _pack_body(x_ref, o_ref):
    t = jnp.swapaxes(x_ref[...], 0, 1)
    d = jnp.concatenate([t, t], axis=1).astype(jnp.bfloat16)
    o_ref[...] = pltpu.bitcast(d, jnp.int32)


def _make_packer(n_rows):
    return pl.pallas_call(
        _pack_body,
        grid=(pl.cdiv(n_rows, PBLK),),
        in_specs=[pl.BlockSpec((E, PBLK), lambda i: (0, i))],
        out_specs=pl.BlockSpec((PBLK // 2, PW), lambda i: (i, 0)),
        out_shape=jax.ShapeDtypeStruct((n_rows // 2, PW), jnp.int32),
    )


_pack_user = _make_packer(1000000)
_pack_movie = _make_packer(100000)


def _gather_body(pid_hbm, tab_hbm, out_hbm, idx_v, rows_v, sem):
    wid = lax.axis_index("s") * NC + lax.axis_index("c")
    base = wid * BPW
    pltpu.sync_copy(pid_hbm.at[pl.ds(base, BPW)], idx_v)
    copies = [
        pltpu.async_copy(tab_hbm.at[idx_v.at[pl.ds(j * CHUNK, CHUNK)]],
                         rows_v.at[pl.ds(j * CHUNK, CHUNK)], sem)
        for j in range(NCHUNK)
    ]
    for c in copies:
        c.wait()
    pltpu.sync_copy(rows_v, out_hbm.at[pl.ds(base, BPW)])


def _make_gather():
    return functools.partial(
        pl.kernel,
        out_type=jax.ShapeDtypeStruct((B, PW), jnp.int32),
        mesh=_sc_mesh,
        scratch_types=[
            pltpu.VMEM((BPW,), jnp.int32),
            pltpu.VMEM((BPW, PW), jnp.int32),
            pltpu.SemaphoreType.DMA,
        ],
    )(_gather_body)


_gather_user = _make_gather()
_gather_movie = _make_gather()


BB = 2048
NBLK = B // BB
_HI = -65536


def _unpack_half(words, parity_is_odd):
    bits = jnp.where(parity_is_odd, words & _HI, words << 16)
    return lax.bitcast_convert_type(bits, jnp.float32)


def _mlp_body(u2_ref, m2_ref, uid_ref, mid_ref, w1a_ref, w1b_ref, b1_ref,
              w2_ref, b2_ref, w3_ref, b3_ref, out_ref):
    upar = (uid_ref[...] % 2)[:, None] == 1
    mpar = (mid_ref[...] % 2)[:, None] == 1
    u = _unpack_half(u2_ref[...][:, :E], upar)
    m = _unpack_half(m2_ref[...][:, :E], mpar)
    h = jnp.dot(u, w1a_ref[...], preferred_element_type=jnp.float32)
    h = h + jnp.dot(m, w1b_ref[...], preferred_element_type=jnp.float32)
    h = jnp.maximum(h + b1_ref[...], 0.0)
    h = jnp.dot(h, w2_ref[...], preferred_element_type=jnp.float32)
    h = jnp.maximum(h + b2_ref[...], 0.0)
    out_ref[...] = jnp.sum(h * w3_ref[...], axis=1) + b3_ref[0]


_mlp = pl.pallas_call(
    _mlp_body,
    grid=(NBLK,),
    in_specs=[
        pl.BlockSpec((BB, PW), lambda i: (i, 0)),
        pl.BlockSpec((BB, PW), lambda i: (i, 0)),
        pl.BlockSpec((BB,), lambda i: (i,)),
        pl.BlockSpec((BB,), lambda i: (i,)),
        pl.BlockSpec((E, H1), lambda i: (0, 0)),
        pl.BlockSpec((E, H1), lambda i: (0, 0)),
        pl.BlockSpec((1, H1), lambda i: (0, 0)),
        pl.BlockSpec((H1, H2), lambda i: (0, 0)),
        pl.BlockSpec((1, H2), lambda i: (0, 0)),
        pl.BlockSpec((1, H2), lambda i: (0, 0)),
        pl.BlockSpec(memory_space=pltpu.SMEM),
    ],
    out_specs=pl.BlockSpec((BB,), lambda i: (i,)),
    out_shape=jax.ShapeDtypeStruct((B,), jnp.float32),
)


def kernel(user_ids, movie_ids, user_table, movie_table, W1, b1, W2, b2, W3, b3):
    uids = user_ids.astype(jnp.int32)
    mids = movie_ids.astype(jnp.int32)
    mpak = _pack_movie(movie_table.T)
    m2 = _gather_movie(mids >> 1, mpak)
    upak = _pack_user(user_table.T)
    u2 = _gather_user(uids >> 1, upak)
    return _mlp(u2, m2, uids, mids, W1[:E], W1[E:], b1.reshape(1, H1), W2,
                b2.reshape(1, H2), W3.reshape(1, H2), b3)


# speedup vs baseline: 2.8987x; 1.1624x over previous
import functools

import jax
import jax.numpy as jnp
from jax import lax
from jax.experimental import pallas as pl
from jax.experimental.pallas import tpu as pltpu
from jax.experimental.pallas import tpu_sc as plsc

B = 16384
E = 64
H1 = 256
H2 = 128
NC = 2
NS = 16
NW = NC * NS
BPW = B // NW
CHUNK = 128
NCHUNK = BPW // CHUNK
PW = 2 * E
PBLK = 16384

_sc_mesh = plsc.VectorSubcoreMesh(core_axis_name="c", subcore_axis_name="s")


def _pack_body(x_ref, o_ref):
    t = jnp.swapaxes(x_ref[...], 0, 1)
    d = jnp.concatenate([t, t], axis=1).astype(jnp.bfloat16)
    o_ref[...] = pltpu.bitcast(d, jnp.int32)


def _make_packer(n_rows):
    return pl.pallas_call(
        _pack_body,
        grid=(pl.cdiv(n_rows, PBLK),),
        in_specs=[pl.BlockSpec((E, PBLK), lambda i: (0, i))],
        out_specs=pl.BlockSpec((PBLK // 2, PW), lambda i: (i, 0)),
        out_shape=jax.ShapeDtypeStruct((n_rows // 2, PW), jnp.int32),
    )


_pack_user = _make_packer(1000000)
_pack_movie = _make_packer(100000)


def _gather_body(pid_hbm, tab_hbm, out_hbm, idx_v, rows_v, sem):
    wid = lax.axis_index("s") * NC + lax.axis_index("c")
    base = wid * BPW
    pltpu.sync_copy(pid_hbm.at[pl.ds(base, BPW)], idx_v)
    copies = [
        pltpu.async_copy(tab_hbm.at[idx_v.at[pl.ds(j * CHUNK, CHUNK)]],
                         rows_v.at[pl.ds(j * CHUNK, CHUNK)], sem)
        for j in range(NCHUNK)
    ]
    for c in copies:
        c.wait()
    pltpu.sync_copy(rows_v, out_hbm.at[pl.ds(base, BPW)])


def _make_gather():
    return functools.partial(
        pl.kernel,
        out_type=jax.ShapeDtypeStruct((B, PW), jnp.int32),
        mesh=_sc_mesh,
        scratch_types=[
            pltpu.VMEM((BPW,), jnp.int32),
            pltpu.VMEM((BPW, PW), jnp.int32),
            pltpu.SemaphoreType.DMA,
        ],
    )(_gather_body)


_gather_user = _make_gather()
_gather_movie = _make_gather()


BB = 4096
NBLK = B // BB
_HI = -65536


def _unpack_half(words, parity_is_odd):
    bits = jnp.where(parity_is_odd, words & _HI, words << 16)
    return lax.bitcast_convert_type(bits, jnp.float32)


def _mlp_body(u2_ref, m2_ref, uid_ref, mid_ref, w1a_ref, w1b_ref, b1_ref,
              w2_ref, b2_ref, w3_ref, b3_ref, out_ref):
    upar = (uid_ref[...] % 2)[:, None] == 1
    mpar = (mid_ref[...] % 2)[:, None] == 1
    u = _unpack_half(u2_ref[...][:, :E], upar)
    m = _unpack_half(m2_ref[...][:, :E], mpar)
    h = jnp.dot(u, w1a_ref[...], preferred_element_type=jnp.float32)
    h = h + jnp.dot(m, w1b_ref[...], preferred_element_type=jnp.float32)
    h = jnp.maximum(h + b1_ref[...], 0.0)
    h = jnp.dot(h, w2_ref[...], preferred_element_type=jnp.float32)
    h = jnp.maximum(h + b2_ref[...], 0.0)
    out_ref[...] = jnp.sum(h * w3_ref[...], axis=1) + b3_ref[0]


_mlp = pl.pallas_call(
    _mlp_body,
    grid=(NBLK,),
    in_specs=[
        pl.BlockSpec((BB, PW), lambda i: (i, 0)),
        pl.BlockSpec((BB, PW), lambda i: (i, 0)),
        pl.BlockSpec((BB,), lambda i: (i,)),
        pl.BlockSpec((BB,), lambda i: (i,)),
        pl.BlockSpec((E, H1), lambda i: (0, 0)),
        pl.BlockSpec((E, H1), lambda i: (0, 0)),
        pl.BlockSpec((1, H1), lambda i: (0, 0)),
        pl.BlockSpec((H1, H2), lambda i: (0, 0)),
        pl.BlockSpec((1, H2), lambda i: (0, 0)),
        pl.BlockSpec((1, H2), lambda i: (0, 0)),
        pl.BlockSpec(memory_space=pltpu.SMEM),
    ],
    out_specs=pl.BlockSpec((BB,), lambda i: (i,)),
    out_shape=jax.ShapeDtypeStruct((B,), jnp.float32),
)


def kernel(user_ids, movie_ids, user_table, movie_table, W1, b1, W2, b2, W3, b3):
    uids = user_ids.astype(jnp.int32)
    mids = movie_ids.astype(jnp.int32)
    mpak = _pack_movie(movie_table.T)
    m2 = _gather_movie(mids >> 1, mpak)
    upak = _pack_user(user_table.T)
    u2 = _gather_user(uids >> 1, upak)
    return _mlp(u2, m2, uids, mids, W1[:E], W1[E:], b1.reshape(1, H1), W2,
                b2.reshape(1, H2), W3.reshape(1, H2), b3)


# speedup vs baseline: 3.1333x; 1.0809x over previous
import functools

import jax
import jax.numpy as jnp
from jax import lax
from jax.experimental import pallas as pl
from jax.experimental.pallas import tpu as pltpu
from jax.experimental.pallas import tpu_sc as plsc

B = 16384
E = 64
H1 = 256
H2 = 128
NC = 2
NS = 16
NW = NC * NS
BPW = B // NW
CHUNK = 128
NCHUNK = BPW // CHUNK
PW = 2 * E
PBLK = 32768

_sc_mesh = plsc.VectorSubcoreMesh(core_axis_name="c", subcore_axis_name="s")


def _pack_body(x_ref, o_ref):
    t = jnp.swapaxes(x_ref[...], 0, 1)
    d = jnp.concatenate([t, t], axis=1).astype(jnp.bfloat16)
    o_ref[...] = pltpu.bitcast(d, jnp.int32)


def _make_packer(n_rows):
    return pl.pallas_call(
        _pack_body,
        grid=(pl.cdiv(n_rows, PBLK),),
        in_specs=[pl.BlockSpec((E, PBLK), lambda i: (0, i))],
        out_specs=pl.BlockSpec((PBLK // 2, PW), lambda i: (i, 0)),
        out_shape=jax.ShapeDtypeStruct((n_rows // 2, PW), jnp.int32),
    )


_pack_user = _make_packer(1000000)
_pack_movie = _make_packer(100000)


def _gather_body(pid_hbm, tab_hbm, out_hbm, idx_v, rows_v, sem):
    wid = lax.axis_index("s") * NC + lax.axis_index("c")
    base = wid * BPW
    pltpu.sync_copy(pid_hbm.at[pl.ds(base, BPW)], idx_v)
    copies = [
        pltpu.async_copy(tab_hbm.at[idx_v.at[pl.ds(j * CHUNK, CHUNK)]],
                         rows_v.at[pl.ds(j * CHUNK, CHUNK)], sem)
        for j in range(NCHUNK)
    ]
    for c in copies:
        c.wait()
    pltpu.sync_copy(rows_v, out_hbm.at[pl.ds(base, BPW)])


def _make_gather():
    return functools.partial(
        pl.kernel,
        out_type=jax.ShapeDtypeStruct((B, PW), jnp.int32),
        mesh=_sc_mesh,
        scratch_types=[
            pltpu.VMEM((BPW,), jnp.int32),
            pltpu.VMEM((BPW, PW), jnp.int32),
            pltpu.SemaphoreType.DMA,
        ],
    )(_gather_body)


_gather_user = _make_gather()
_gather_movie = _make_gather()


BB = 8192
NBLK = B // BB
_HI = -65536


def _unpack_half(words, parity_is_odd):
    bits = jnp.where(parity_is_odd, words & _HI, words << 16)
    return lax.bitcast_convert_type(bits, jnp.float32)


def _mlp_body(u2_ref, m2_ref, uid_ref, mid_ref, w1a_ref, w1b_ref, b1_ref,
              w2_ref, b2_ref, w3_ref, b3_ref, out_ref):
    upar = (uid_ref[...] % 2)[:, None] == 1
    mpar = (mid_ref[...] % 2)[:, None] == 1
    u = _unpack_half(u2_ref[...][:, :E], upar)
    m = _unpack_half(m2_ref[...][:, :E], mpar)
    h = jnp.dot(u, w1a_ref[...], preferred_element_type=jnp.float32)
    h = h + jnp.dot(m, w1b_ref[...], preferred_element_type=jnp.float32)
    h = jnp.maximum(h + b1_ref[...], 0.0)
    h = jnp.dot(h, w2_ref[...], preferred_element_type=jnp.float32)
    h = jnp.maximum(h + b2_ref[...], 0.0)
    out_ref[...] = jnp.sum(h * w3_ref[...], axis=1) + b3_ref[0]


_mlp = pl.pallas_call(
    _mlp_body,
    grid=(NBLK,),
    in_specs=[
        pl.BlockSpec((BB, PW), lambda i: (i, 0)),
        pl.BlockSpec((BB, PW), lambda i: (i, 0)),
        pl.BlockSpec((BB,), lambda i: (i,)),
        pl.BlockSpec((BB,), lambda i: (i,)),
        pl.BlockSpec((E, H1), lambda i: (0, 0)),
        pl.BlockSpec((E, H1), lambda i: (0, 0)),
        pl.BlockSpec((1, H1), lambda i: (0, 0)),
        pl.BlockSpec((H1, H2), lambda i: (0, 0)),
        pl.BlockSpec((1, H2), lambda i: (0, 0)),
        pl.BlockSpec((1, H2), lambda i: (0, 0)),
        pl.BlockSpec(memory_space=pltpu.SMEM),
    ],
    out_specs=pl.BlockSpec((BB,), lambda i: (i,)),
    out_shape=jax.ShapeDtypeStruct((B,), jnp.float32),
)


def kernel(user_ids, movie_ids, user_table, movie_table, W1, b1, W2, b2, W3, b3):
    uids = user_ids.astype(jnp.int32)
    mids = movie_ids.astype(jnp.int32)
    mpak = _pack_movie(movie_table.T)
    m2 = _gather_movie(mids >> 1, mpak)
    upak = _pack_user(user_table.T)
    u2 = _gather_user(uids >> 1, upak)
    return _mlp(u2, m2, uids, mids, W1[:E], W1[E:], b1.reshape(1, H1), W2,
                b2.reshape(1, H2), W3.reshape(1, H2), b3)
